# baseline (speedup 1.0000x reference)
import functools

import jax
import jax.numpy as jnp
from jax import lax
from jax.experimental import pallas as pl
from jax.experimental.pallas import tpu as pltpu
from jax.experimental.pallas import tpu_sc as plsc

B, H, W = 4, 512, 512
N_TOTAL = B * H * W
NC, NS, L = 2, 16, 16
NW = NC * NS
PER_W = N_TOTAL // NW
RW = PER_W // W
NVR = W // L

_MESH = plsc.VectorSubcoreMesh(
    core_axis_name="c", subcore_axis_name="s", num_cores=NC, num_subcores=NS
)


@functools.partial(
    pl.kernel,
    out_type=jax.ShapeDtypeStruct((NW, 3 * L), jnp.float32),
    mesh=_MESH,
    scratch_types=[
        pltpu.VMEM((2, RW // 8, W), jnp.float32),
        pltpu.VMEM((2, RW // 8, W), jnp.float32),
        pltpu.VMEM((2, RW // 8, W), jnp.int32),
        pltpu.VMEM((3 * L,), jnp.float32),
        pltpu.SemaphoreType.DMA,
        pltpu.SemaphoreType.DMA,
    ],
)
def _main_pass(pred_hbm, gt_hbm, mask_hbm, out_hbm, pv, gv, mv, accv,
               sem0, sem1):
    c = lax.axis_index("c")
    s = lax.axis_index("s")
    wid = c * NS + s
    b = wid // 8
    r0 = (wid % 8) * RW
    chr_ = RW // 8
    sems = (sem0, sem1)

    def start(rnd):
        bf = rnd & 1
        lo = r0 + rnd * chr_
        return (
            pltpu.async_copy(pred_hbm.at[b, 0, pl.ds(lo, chr_)],
                             pv.at[bf], sems[bf]),
            pltpu.async_copy(gt_hbm.at[b, pl.ds(lo, chr_)],
                             gv.at[bf], sems[bf]),
            pltpu.async_copy(mask_hbm.at[b, pl.ds(lo, chr_)],
                             mv.at[bf], sems[bf]),
        )

    def accumulate(bf, carry):
        @plsc.parallel_loop(0, chr_ * W // L, carry=carry, unroll=16)
        def acc(i, carry):
            ls, ps, pc = carry
            r = i // NVR
            col = (i % NVR) * L
            p = pv[bf, r, pl.ds(col, L)]
            g = gv[bf, r, pl.ds(col, L)]
            m = mv[bf, r, pl.ds(col, L)].astype(jnp.float32)
            loss = jnp.abs(p - g)
            return (ls + loss, ps + loss * m, pc + m)

        return acc

    z = jnp.zeros((L,), jnp.float32)
    carry = (z, z, z)
    d = start(0)
    for rnd in range(8):
        for x in d:
            x.wait()
        if rnd < 7:
            d = start(rnd + 1)
        carry = accumulate(rnd & 1, carry)
    ls, ps, pc = carry
    accv[pl.ds(0, L)] = ls
    accv[pl.ds(L, L)] = ps
    accv[pl.ds(2 * L, L)] = pc
    pltpu.sync_copy(accv, out_hbm.at[wid])


_CHR = 16


def _make_hist_pass(shift, nbits, pshift):
    nbins = 1 << nbits

    @functools.partial(
        pl.kernel,
        out_type=(
            jax.ShapeDtypeStruct((NW, nbins * L), jnp.float32),
            jax.ShapeDtypeStruct((NW, nbins * L), jnp.float32),
        ),
        mesh=_MESH,
        compiler_params=pltpu.CompilerParams(needs_layout_passes=False),
        scratch_types=[
            pltpu.VMEM((_CHR, W), jnp.float32),
            pltpu.VMEM((_CHR, W), jnp.float32),
            pltpu.VMEM((_CHR, W), jnp.int32),
            pltpu.VMEM((L,), jnp.int32),
            pltpu.VMEM((nbins * L,), jnp.float32),
            pltpu.VMEM((nbins * L,), jnp.float32),
        ],
    )
    def hist(pred_hbm, gt_hbm, mask_hbm, pfx_hbm, cnt_hbm, sum_hbm,
             pv, gv, mv, pfxv, hcnt, hsum):
        c = lax.axis_index("c")
        s = lax.axis_index("s")
        wid = c * NS + s
        b = wid // 8
        r0 = (wid % 8) * RW
        pltpu.sync_copy(pfx_hbm, pfxv)
        pfx = pfxv[pl.ds(0, L)]

        zv = jnp.zeros((L,), jnp.float32)

        def zero_body(i, _):
            hcnt[pl.ds(i * L, L)] = zv
            hsum[pl.ds(i * L, L)] = zv
            return 0

        lax.fori_loop(0, nbins, zero_body, 0)

        lanes = lax.iota(jnp.int32, L)
        ones = jnp.ones((L,), jnp.float32)

        for j in range(RW // _CHR):
            pltpu.sync_copy(pred_hbm.at[b, 0, pl.ds(r0 + j * _CHR, _CHR)], pv)
            pltpu.sync_copy(gt_hbm.at[b, pl.ds(r0 + j * _CHR, _CHR)], gv)
            pltpu.sync_copy(mask_hbm.at[b, pl.ds(r0 + j * _CHR, _CHR)], mv)

            def body(i, _):
                r = i // NVR
                col = (i % NVR) * L
                p = pv[r, pl.ds(col, L)]
                g = gv[r, pl.ds(col, L)]
                m = mv[r, pl.ds(col, L)]
                loss = jnp.abs(p - g)
                bits = lax.bitcast_convert_type(loss, jnp.int32)
                match = (m == 0) & (lax.shift_right_logical(bits, pshift) == pfx)
                idx = ((lax.shift_right_logical(bits, shift) & (nbins - 1)) * L
                       + lanes)
                plsc.addupdate_scatter(hcnt, [idx], ones, mask=match)
                plsc.addupdate_scatter(hsum, [idx], loss, mask=match)
                return 0

            lax.fori_loop(0, _CHR * W // L, body, 0)

        pltpu.sync_copy(hcnt, cnt_hbm.at[wid])
        pltpu.sync_copy(hsum, sum_hbm.at[wid])

    return hist


_HIST_LEVELS = (
    (20, 11, 31),
    (9, 11, 20),
    (0, 9, 9),
)
_HIST_PASSES = tuple(_make_hist_pass(*lvl) for lvl in _HIST_LEVELS)


def _topk_neg_sum(pred, gt, mask, k):
    kf = k.astype(jnp.float32)
    sum_above = jnp.float32(0.0)
    cnt_above = jnp.float32(0.0)
    prefix = jnp.zeros((L,), jnp.int32)
    zero1 = jnp.zeros((1,), jnp.float32)
    for (shift, nbits, pshift), hist in zip(_HIST_LEVELS, _HIST_PASSES):
        nbins = 1 << nbits
        cnts, sums = hist(pred, gt, mask, prefix)
        cnt = cnts.reshape(NW, nbins, L).sum(axis=(0, 2))
        sm = sums.reshape(NW, nbins, L).sum(axis=(0, 2))
        cc = jnp.cumsum(cnt[::-1])[::-1]
        cs = jnp.cumsum(sm[::-1])[::-1]
        ccp = jnp.concatenate([cc, zero1])
        csp = jnp.concatenate([cs, zero1])
        kk = kf - cnt_above
        b = jnp.sum((cc >= kk).astype(jnp.int32)) - 1
        cnt_above = cnt_above + ccp[b + 1]
        sum_above = sum_above + csp[b + 1]
        prefix = jnp.full((L,), (prefix[0] << nbits) | b, jnp.int32)
    thresh = lax.bitcast_convert_type(prefix[0], jnp.float32)
    return jnp.where(kf > 0, sum_above + (kf - cnt_above) * thresh, 0.0)


def kernel(pred, gt, mask):
    parts = _main_pass(pred, gt, mask)
    sums = parts.reshape(NW, 3, L).sum(axis=(0, 2))
    loss_sum, pos_sum, pos_cntf = sums[0], sums[1], sums[2]

    pos_cnt = pos_cntf.astype(jnp.int32)
    neg_cnt_all = jnp.int32(N_TOTAL) - pos_cnt
    neg_cnt = jnp.minimum(neg_cnt_all, (pos_cntf * 3.0).astype(jnp.int32))
    neg_sum_all = loss_sum - pos_sum

    neg_top = lax.cond(
        neg_cnt >= neg_cnt_all,
        lambda: neg_sum_all,
        lambda: _topk_neg_sum(pred, gt, mask, neg_cnt),
    )

    pos_loss = pos_sum / pos_cnt
    neg_loss = neg_top / neg_cnt
    total = pos_loss + neg_loss
    return (total, pos_loss, neg_loss)


# speedup vs baseline: 1.0169x; 1.0169x over previous
import functools

import jax
import jax.numpy as jnp
from jax import lax
from jax.experimental import pallas as pl
from jax.experimental.pallas import tpu as pltpu
from jax.experimental.pallas import tpu_sc as plsc

B, H, W = 4, 512, 512
N_TOTAL = B * H * W
NC, NS, L = 2, 16, 16
NW = NC * NS
PER_W = N_TOTAL // NW
RW = PER_W // W
NVR = W // L

_MESH = plsc.VectorSubcoreMesh(
    core_axis_name="c", subcore_axis_name="s", num_cores=NC, num_subcores=NS
)


@functools.partial(
    pl.kernel,
    out_type=jax.ShapeDtypeStruct((NW, 3 * L), jnp.float32),
    mesh=_MESH,
    scratch_types=[
        pltpu.VMEM((2, RW // 4, W), jnp.float32),
        pltpu.VMEM((2, RW // 4, W), jnp.float32),
        pltpu.VMEM((2, RW // 4, W), jnp.int32),
        pltpu.VMEM((3 * L,), jnp.float32),
        pltpu.SemaphoreType.DMA,
        pltpu.SemaphoreType.DMA,
    ],
)
def _main_pass(pred_hbm, gt_hbm, mask_hbm, out_hbm, pv, gv, mv, accv,
               sem0, sem1):
    c = lax.axis_index("c")
    s = lax.axis_index("s")
    wid = c * NS + s
    b = wid // 8
    r0 = (wid % 8) * RW
    chr_ = RW // 4
    sems = (sem0, sem1)

    def start(rnd):
        bf = rnd & 1
        lo = r0 + rnd * chr_
        return (
            pltpu.async_copy(pred_hbm.at[b, 0, pl.ds(lo, chr_)],
                             pv.at[bf], sems[bf]),
            pltpu.async_copy(gt_hbm.at[b, pl.ds(lo, chr_)],
                             gv.at[bf], sems[bf]),
            pltpu.async_copy(mask_hbm.at[b, pl.ds(lo, chr_)],
                             mv.at[bf], sems[bf]),
        )

    def accumulate(bf, carry):
        @plsc.parallel_loop(0, chr_ * W // L, carry=carry, unroll=8)
        def acc(i, carry):
            ls, ps, pc = carry
            r = i // NVR
            col = (i % NVR) * L
            p = pv[bf, r, pl.ds(col, L)]
            g = gv[bf, r, pl.ds(col, L)]
            m = mv[bf, r, pl.ds(col, L)].astype(jnp.float32)
            loss = jnp.abs(p - g)
            return (ls + loss, ps + loss * m, pc + m)

        return acc

    z = jnp.zeros((L,), jnp.float32)
    carry = (z, z, z)
    d = start(0)
    for rnd in range(4):
        for x in d:
            x.wait()
        if rnd < 3:
            d = start(rnd + 1)
        carry = accumulate(rnd & 1, carry)
    ls, ps, pc = carry
    accv[pl.ds(0, L)] = ls
    accv[pl.ds(L, L)] = ps
    accv[pl.ds(2 * L, L)] = pc
    pltpu.sync_copy(accv, out_hbm.at[wid])


_CHR = 16


def _make_hist_pass(shift, nbits, pshift):
    nbins = 1 << nbits

    @functools.partial(
        pl.kernel,
        out_type=(
            jax.ShapeDtypeStruct((NW, nbins * L), jnp.float32),
            jax.ShapeDtypeStruct((NW, nbins * L), jnp.float32),
        ),
        mesh=_MESH,
        compiler_params=pltpu.CompilerParams(needs_layout_passes=False),
        scratch_types=[
            pltpu.VMEM((_CHR, W), jnp.float32),
            pltpu.VMEM((_CHR, W), jnp.float32),
            pltpu.VMEM((_CHR, W), jnp.int32),
            pltpu.VMEM((L,), jnp.int32),
            pltpu.VMEM((nbins * L,), jnp.float32),
            pltpu.VMEM((nbins * L,), jnp.float32),
        ],
    )
    def hist(pred_hbm, gt_hbm, mask_hbm, pfx_hbm, cnt_hbm, sum_hbm,
             pv, gv, mv, pfxv, hcnt, hsum):
        c = lax.axis_index("c")
        s = lax.axis_index("s")
        wid = c * NS + s
        b = wid // 8
        r0 = (wid % 8) * RW
        pltpu.sync_copy(pfx_hbm, pfxv)
        pfx = pfxv[pl.ds(0, L)]

        zv = jnp.zeros((L,), jnp.float32)

        def zero_body(i, _):
            hcnt[pl.ds(i * L, L)] = zv
            hsum[pl.ds(i * L, L)] = zv
            return 0

        lax.fori_loop(0, nbins, zero_body, 0)

        lanes = lax.iota(jnp.int32, L)
        ones = jnp.ones((L,), jnp.float32)

        for j in range(RW // _CHR):
            pltpu.sync_copy(pred_hbm.at[b, 0, pl.ds(r0 + j * _CHR, _CHR)], pv)
            pltpu.sync_copy(gt_hbm.at[b, pl.ds(r0 + j * _CHR, _CHR)], gv)
            pltpu.sync_copy(mask_hbm.at[b, pl.ds(r0 + j * _CHR, _CHR)], mv)

            def body(i, _):
                r = i // NVR
                col = (i % NVR) * L
                p = pv[r, pl.ds(col, L)]
                g = gv[r, pl.ds(col, L)]
                m = mv[r, pl.ds(col, L)]
                loss = jnp.abs(p - g)
                bits = lax.bitcast_convert_type(loss, jnp.int32)
                match = (m == 0) & (lax.shift_right_logical(bits, pshift) == pfx)
                idx = ((lax.shift_right_logical(bits, shift) & (nbins - 1)) * L
                       + lanes)
                plsc.addupdate_scatter(hcnt, [idx], ones, mask=match)
                plsc.addupdate_scatter(hsum, [idx], loss, mask=match)
                return 0

            lax.fori_loop(0, _CHR * W // L, body, 0)

        pltpu.sync_copy(hcnt, cnt_hbm.at[wid])
        pltpu.sync_copy(hsum, sum_hbm.at[wid])

    return hist


_HIST_LEVELS = (
    (20, 11, 31),
    (9, 11, 20),
    (0, 9, 9),
)
_HIST_PASSES = tuple(_make_hist_pass(*lvl) for lvl in _HIST_LEVELS)


def _topk_neg_sum(pred, gt, mask, k):
    kf = k.astype(jnp.float32)
    sum_above = jnp.float32(0.0)
    cnt_above = jnp.float32(0.0)
    prefix = jnp.zeros((L,), jnp.int32)
    zero1 = jnp.zeros((1,), jnp.float32)
    for (shift, nbits, pshift), hist in zip(_HIST_LEVELS, _HIST_PASSES):
        nbins = 1 << nbits
        cnts, sums = hist(pred, gt, mask, prefix)
        cnt = cnts.reshape(NW, nbins, L).sum(axis=(0, 2))
        sm = sums.reshape(NW, nbins, L).sum(axis=(0, 2))
        cc = jnp.cumsum(cnt[::-1])[::-1]
        cs = jnp.cumsum(sm[::-1])[::-1]
        ccp = jnp.concatenate([cc, zero1])
        csp = jnp.concatenate([cs, zero1])
        kk = kf - cnt_above
        b = jnp.sum((cc >= kk).astype(jnp.int32)) - 1
        cnt_above = cnt_above + ccp[b + 1]
        sum_above = sum_above + csp[b + 1]
        prefix = jnp.full((L,), (prefix[0] << nbits) | b, jnp.int32)
    thresh = lax.bitcast_convert_type(prefix[0], jnp.float32)
    return jnp.where(kf > 0, sum_above + (kf - cnt_above) * thresh, 0.0)


def kernel(pred, gt, mask):
    parts = _main_pass(pred, gt, mask)
    sums = parts.reshape(NW, 3, L).sum(axis=(0, 2))
    loss_sum, pos_sum, pos_cntf = sums[0], sums[1], sums[2]

    pos_cnt = pos_cntf.astype(jnp.int32)
    neg_cnt_all = jnp.int32(N_TOTAL) - pos_cnt
    neg_cnt = jnp.minimum(neg_cnt_all, (pos_cntf * 3.0).astype(jnp.int32))
    neg_sum_all = loss_sum - pos_sum

    neg_top = lax.cond(
        neg_cnt >= neg_cnt_all,
        lambda: neg_sum_all,
        lambda: _topk_neg_sum(pred, gt, mask, neg_cnt),
    )

    pos_loss = pos_sum / pos_cnt
    neg_loss = neg_top / neg_cnt
    total = pos_loss + neg_loss
    return (total, pos_loss, neg_loss)
